# baseline (speedup 1.0000x reference)
import jax
import jax.numpy as jnp
from jax.experimental import pallas as pl


def _add_body(x_ref, p_ref, o_ref):
    o_ref[...] = x_ref[...] + p_ref[...]


def kernel(x, pos_table):
    B, P, D = x.shape
    F = P * D
    x2 = x.reshape(B, F)
    p2 = pos_table.reshape(1, F)
    BB = 8
    out = pl.pallas_call(
        _add_body,
        grid=(B // BB,),
        in_specs=[
            pl.BlockSpec((BB, F), lambda i: (i, 0)),
            pl.BlockSpec((1, F), lambda i: (0, 0)),
        ],
        out_specs=pl.BlockSpec((BB, F), lambda i: (i, 0)),
        out_shape=jax.ShapeDtypeStruct((B, F), x.dtype),
    )(x2, p2)
    return out.reshape(B, P, D)
